# baseline (speedup 1.0000x reference)
import functools

import jax
import jax.numpy as jnp
from jax import lax
from jax.experimental import pallas as pl
from jax.experimental.pallas import tpu as pltpu
from jax.experimental.pallas import tpu_sc as plsc

NUM_EXPERTS = 8
TOP_K = 2
D_MODEL = 2048
D_FF = 768
N_TOKENS = 2048
N_ASSIGN = N_TOKENS * TOP_K
TM = 128
NT = N_ASSIGN // TM + NUM_EXPERTS
PAD_TOTAL = NT * TM

NW = 32
DISPATCH_CHUNK = 32
COMBINE_CHUNK = 16

_NEG = -1e30


def _router_body(x_ref, gwt_ref, ids_ref, w_ref):
    logits = jnp.dot(x_ref[...], gwt_ref[...], preferred_element_type=jnp.float32)
    lane = lax.broadcasted_iota(jnp.int32, logits.shape, 1)
    logits = jnp.where(lane < NUM_EXPERTS, logits, _NEG)
    m1 = jnp.max(logits, axis=1, keepdims=True)
    a1 = jnp.min(jnp.where(logits == m1, lane, 127), axis=1, keepdims=True)
    l2 = jnp.where(lane == a1, _NEG, logits)
    m2 = jnp.max(l2, axis=1, keepdims=True)
    a2 = jnp.min(jnp.where(l2 == m2, lane, 127), axis=1, keepdims=True)
    w1 = 1.0 / (1.0 + jnp.exp(m2 - m1))
    w2 = 1.0 - w1
    ids_ref[...] = jnp.where(lane == 0, a1, jnp.where(lane == 1, a2, 0))
    w_ref[...] = jnp.where(lane == 0, w1, jnp.where(lane == 1, w2, 0.0))


def _run_router(x, gate_w):
    gwt = jnp.zeros((D_MODEL, 128), jnp.float32).at[:, :NUM_EXPERTS].set(gate_w.T)
    rt = 256
    ids, w = pl.pallas_call(
        _router_body,
        grid=(N_TOKENS // rt,),
        in_specs=[
            pl.BlockSpec((rt, D_MODEL), lambda i: (i, 0)),
            pl.BlockSpec((D_MODEL, 128), lambda i: (0, 0)),
        ],
        out_specs=[
            pl.BlockSpec((rt, 128), lambda i: (i, 0)),
            pl.BlockSpec((rt, 128), lambda i: (i, 0)),
        ],
        out_shape=[
            jax.ShapeDtypeStruct((N_TOKENS, 128), jnp.int32),
            jax.ShapeDtypeStruct((N_TOKENS, 128), jnp.float32),
        ],
    )(x, gwt)
    return ids[:, :TOP_K], w[:, :TOP_K]


def _build_metadata(topk_ids):
    flat_e = topk_ids.reshape(-1)
    onehot = (flat_e[:, None] == jnp.arange(NUM_EXPERTS)[None, :]).astype(jnp.int32)
    csum = jnp.cumsum(onehot, axis=0)
    counts = csum[-1]
    ranks = jnp.take_along_axis(csum, flat_e[:, None], axis=1)[:, 0] - 1
    padded_counts = ((counts + TM - 1) // TM) * TM
    pad_end = jnp.cumsum(padded_counts)
    pad_off = pad_end - padded_counts
    dst = pad_off[flat_e] + ranks
    src_token = jnp.zeros((PAD_TOTAL,), jnp.int32).at[dst].set(
        jnp.arange(N_ASSIGN, dtype=jnp.int32) // TOP_K)
    tile_start = jnp.arange(NT, dtype=jnp.int32) * TM
    tile_expert = jnp.searchsorted(pad_end, tile_start, side="right").astype(jnp.int32)
    tile_valid = (tile_start < pad_end[-1]).astype(jnp.int32)
    last_e = jnp.max(jnp.where(counts > 0, jnp.arange(NUM_EXPERTS, dtype=jnp.int32), 0))
    tile_expert = jnp.where(tile_valid == 1, jnp.minimum(tile_expert, NUM_EXPERTS - 1), last_e)
    return src_token, dst, tile_expert, tile_valid


def _dispatch_body(x_hbm, tok_hbm, xs_hbm, idx_v, rows_v, sem):
    wid = lax.axis_index("s") * 2 + lax.axis_index("c")
    per_w = PAD_TOTAL // NW
    base = wid * per_w

    def chunk(i, carry):
        b = base + i * DISPATCH_CHUNK
        pltpu.sync_copy(tok_hbm.at[pl.ds(b, DISPATCH_CHUNK)], idx_v)
        pltpu.async_copy(x_hbm.at[idx_v], rows_v, sem).wait()
        pltpu.sync_copy(rows_v, xs_hbm.at[pl.ds(b, DISPATCH_CHUNK)])
        return carry

    lax.fori_loop(0, per_w // DISPATCH_CHUNK, chunk, 0)


def _run_dispatch(x, src_token):
    mesh = plsc.VectorSubcoreMesh(core_axis_name="c", subcore_axis_name="s")
    f = pl.kernel(
        _dispatch_body,
        out_type=jax.ShapeDtypeStruct((PAD_TOTAL, D_MODEL), jnp.float32),
        mesh=mesh,
        scratch_types=[
            pltpu.VMEM((DISPATCH_CHUNK,), jnp.int32),
            pltpu.VMEM((DISPATCH_CHUNK, D_MODEL), jnp.float32),
            pltpu.SemaphoreType.DMA,
        ],
    )
    return f(x, src_token)


def _mm_body(te_ref, tv_ref, x_ref, wg_ref, wu_ref, wd_ref, y_ref):
    @pl.when(tv_ref[pl.program_id(0)] > 0)
    def _():
        x = x_ref[...]
        g = lax.dot_general(x, wg_ref[0], (((1,), (1,)), ((), ())),
                            preferred_element_type=jnp.float32)
        u = lax.dot_general(x, wu_ref[0], (((1,), (1,)), ((), ())),
                            preferred_element_type=jnp.float32)
        h = g * jax.nn.sigmoid(g) * u
        y_ref[...] = lax.dot_general(h, wd_ref[0], (((1,), (1,)), ((), ())),
                                     preferred_element_type=jnp.float32)


def _run_grouped_mm(x_sorted, w_gate, w_up, w_down, tile_expert, tile_valid):
    grid_spec = pltpu.PrefetchScalarGridSpec(
        num_scalar_prefetch=2,
        grid=(NT,),
        in_specs=[
            pl.BlockSpec((TM, D_MODEL), lambda i, te, tv: (i, 0)),
            pl.BlockSpec((1, D_FF, D_MODEL), lambda i, te, tv: (te[i], 0, 0)),
            pl.BlockSpec((1, D_FF, D_MODEL), lambda i, te, tv: (te[i], 0, 0)),
            pl.BlockSpec((1, D_MODEL, D_FF), lambda i, te, tv: (te[i], 0, 0)),
        ],
        out_specs=pl.BlockSpec((TM, D_MODEL), lambda i, te, tv: (i, 0)),
    )
    return pl.pallas_call(
        _mm_body,
        grid_spec=grid_spec,
        out_shape=jax.ShapeDtypeStruct((PAD_TOTAL, D_MODEL), jnp.float32),
        compiler_params=pltpu.CompilerParams(
            dimension_semantics=("arbitrary",),
            vmem_limit_bytes=100 * 1024 * 1024,
        ),
    )(tile_expert, tile_valid, x_sorted, w_gate, w_up, w_down)


def _combine_body(y_hbm, p0_hbm, p1_hbm, w0_hbm, w1_hbm, out_hbm,
                  i0_v, i1_v, w0_v, w1_v, y0_v, y1_v, ov_v, sem0, sem1):
    wid = lax.axis_index("s") * 2 + lax.axis_index("c")
    per_w = N_TOKENS // NW
    base = wid * per_w
    n_slice = D_MODEL // 16

    def chunk(ci, carry):
        b = base + ci * COMBINE_CHUNK
        pltpu.sync_copy(p0_hbm.at[pl.ds(b, COMBINE_CHUNK)], i0_v)
        pltpu.sync_copy(p1_hbm.at[pl.ds(b, COMBINE_CHUNK)], i1_v)
        pltpu.sync_copy(w0_hbm.at[pl.ds(b, COMBINE_CHUNK)], w0_v)
        pltpu.sync_copy(w1_hbm.at[pl.ds(b, COMBINE_CHUNK)], w1_v)
        cp0 = pltpu.async_copy(y_hbm.at[i0_v], y0_v, sem0)
        cp1 = pltpu.async_copy(y_hbm.at[i1_v], y1_v, sem1)
        cp0.wait()
        cp1.wait()

        def per_token(j, c2):
            wb0 = plsc.load_gather(w0_v, [jnp.full((16,), j, jnp.int32)])
            wb1 = plsc.load_gather(w1_v, [jnp.full((16,), j, jnp.int32)])

            def per_slice(s, c3):
                sl = pl.ds(s * 16, 16)
                ov_v[j, sl] = y0_v[j, sl] * wb0 + y1_v[j, sl] * wb1
                return c3

            lax.fori_loop(0, n_slice, per_slice, 0)
            return c2

        lax.fori_loop(0, COMBINE_CHUNK, per_token, 0)
        pltpu.sync_copy(ov_v, out_hbm.at[pl.ds(b, COMBINE_CHUNK)])
        return carry

    lax.fori_loop(0, per_w // COMBINE_CHUNK, chunk, 0)


def _run_combine(y_sorted, pos, topk_w):
    mesh = plsc.VectorSubcoreMesh(core_axis_name="c", subcore_axis_name="s")
    f = pl.kernel(
        _combine_body,
        out_type=jax.ShapeDtypeStruct((N_TOKENS, D_MODEL), jnp.float32),
        mesh=mesh,
        scratch_types=[
            pltpu.VMEM((COMBINE_CHUNK,), jnp.int32),
            pltpu.VMEM((COMBINE_CHUNK,), jnp.int32),
            pltpu.VMEM((COMBINE_CHUNK,), jnp.float32),
            pltpu.VMEM((COMBINE_CHUNK,), jnp.float32),
            pltpu.VMEM((COMBINE_CHUNK, D_MODEL), jnp.float32),
            pltpu.VMEM((COMBINE_CHUNK, D_MODEL), jnp.float32),
            pltpu.VMEM((COMBINE_CHUNK, D_MODEL), jnp.float32),
            pltpu.SemaphoreType.DMA,
            pltpu.SemaphoreType.DMA,
        ],
        compiler_params=pltpu.CompilerParams(needs_layout_passes=False),
    )
    p0 = pos[:, 0].astype(jnp.int32)
    p1 = pos[:, 1].astype(jnp.int32)
    w0 = topk_w[:, 0]
    w1 = topk_w[:, 1]
    return f(y_sorted, p0, p1, w0, w1)


def kernel(hidden_states, gate_w, w_gate, w_up, w_down):
    B, S, H = hidden_states.shape
    x = hidden_states.reshape(-1, H)
    topk_ids, topk_w = _run_router(x, gate_w)
    src_token, dst, tile_expert, tile_valid = _build_metadata(topk_ids)
    x_sorted = _run_dispatch(x, src_token)
    y_sorted = _run_grouped_mm(x_sorted, w_gate, w_up, w_down, tile_expert, tile_valid)
    out = _run_combine(y_sorted, dst.reshape(N_TOKENS, TOP_K), topk_w)
    return out.reshape(B, S, H)


# speedup vs baseline: 1.3684x; 1.3684x over previous
import functools

import jax
import jax.numpy as jnp
from jax import lax
from jax.experimental import pallas as pl
from jax.experimental.pallas import tpu as pltpu
from jax.experimental.pallas import tpu_sc as plsc

NUM_EXPERTS = 8
TOP_K = 2
D_MODEL = 2048
D_FF = 768
N_TOKENS = 2048
N_ASSIGN = N_TOKENS * TOP_K
TM = 128
NT = N_ASSIGN // TM + NUM_EXPERTS
PAD_TOTAL = NT * TM

NW = 32
DISPATCH_CHUNK = 16
COMBINE_CHUNK = 16

_NEG = -1e30


def _router_body(x_ref, gwt_ref, ids_ref, w_ref):
    logits = jnp.dot(x_ref[...], gwt_ref[...], preferred_element_type=jnp.float32)
    lane = lax.broadcasted_iota(jnp.int32, logits.shape, 1)
    logits = jnp.where(lane < NUM_EXPERTS, logits, _NEG)
    m1 = jnp.max(logits, axis=1, keepdims=True)
    a1 = jnp.min(jnp.where(logits == m1, lane, 127), axis=1, keepdims=True)
    l2 = jnp.where(lane == a1, _NEG, logits)
    m2 = jnp.max(l2, axis=1, keepdims=True)
    a2 = jnp.min(jnp.where(l2 == m2, lane, 127), axis=1, keepdims=True)
    w1 = 1.0 / (1.0 + jnp.exp(m2 - m1))
    w2 = 1.0 - w1
    ids_ref[...] = jnp.where(lane == 0, a1, jnp.where(lane == 1, a2, 0))
    w_ref[...] = jnp.where(lane == 0, w1, jnp.where(lane == 1, w2, 0.0))


def _run_router(x, gate_w):
    gwt = jnp.zeros((D_MODEL, 128), jnp.float32).at[:, :NUM_EXPERTS].set(gate_w.T)
    rt = 256
    ids, w = pl.pallas_call(
        _router_body,
        grid=(N_TOKENS // rt,),
        in_specs=[
            pl.BlockSpec((rt, D_MODEL), lambda i: (i, 0)),
            pl.BlockSpec((D_MODEL, 128), lambda i: (0, 0)),
        ],
        out_specs=[
            pl.BlockSpec((rt, 128), lambda i: (i, 0)),
            pl.BlockSpec((rt, 128), lambda i: (i, 0)),
        ],
        out_shape=[
            jax.ShapeDtypeStruct((N_TOKENS, 128), jnp.int32),
            jax.ShapeDtypeStruct((N_TOKENS, 128), jnp.float32),
        ],
    )(x, gwt)
    return ids[:, :TOP_K], w[:, :TOP_K]


def _build_metadata(topk_ids):
    flat_e = topk_ids.reshape(-1)
    onehot = (flat_e[:, None] == jnp.arange(NUM_EXPERTS)[None, :]).astype(jnp.int32)
    csum = jnp.cumsum(onehot, axis=0)
    counts = csum[-1]
    ranks = jnp.sum(csum * onehot, axis=1) - 1
    padded_counts = ((counts + TM - 1) // TM) * TM
    pad_end = jnp.cumsum(padded_counts)
    pad_off = pad_end - padded_counts
    dst = jnp.sum(onehot * pad_off[None, :], axis=1) + ranks
    tile_start = jnp.arange(NT, dtype=jnp.int32) * TM
    tile_expert = jnp.sum(
        (tile_start[:, None] >= pad_end[None, :]).astype(jnp.int32), axis=1)
    tile_valid = (tile_start < pad_end[-1]).astype(jnp.int32)
    last_e = jnp.max(jnp.where(counts > 0, jnp.arange(NUM_EXPERTS, dtype=jnp.int32), 0))
    tile_expert = jnp.where(tile_valid == 1,
                            jnp.minimum(tile_expert, NUM_EXPERTS - 1), last_e)
    return dst.astype(jnp.int32), tile_expert.astype(jnp.int32), tile_valid


def _dispatch_body(x_hbm, d0_hbm, d1_hbm, xs_hbm,
                  i0a_v, i1a_v, i0b_v, i1b_v, buf_a, buf_b,
                  sem_ra, sem_rb, sem_w):
    wid = lax.axis_index("s") * 2 + lax.axis_index("c")
    per_w = N_TOKENS // NW
    base = wid * per_w
    ch = DISPATCH_CHUNK
    n_chunks = per_w // ch

    def rd(c, buf, sem):
        return pltpu.async_copy(x_hbm.at[pl.ds(base + c * ch, ch)], buf, sem)

    def idx_load(c, i0, i1):
        pltpu.sync_copy(d0_hbm.at[pl.ds(base + c * ch, ch)], i0)
        pltpu.sync_copy(d1_hbm.at[pl.ds(base + c * ch, ch)], i1)

    bufs = (buf_a, buf_b)
    idxs = ((i0a_v, i1a_v), (i0b_v, i1b_v))
    rsems = (sem_ra, sem_rb)

    reads = {0: rd(0, buf_a, sem_ra)}
    if n_chunks > 1:
        reads[1] = rd(1, buf_b, sem_rb)
    scats = {}
    for c in range(n_chunks):
        s = c % 2
        if c >= 2:
            scats[c - 2][0].wait()
            scats[c - 2][1].wait()
            reads[c] = rd(c, bufs[s], rsems[s])
        idx_load(c, *idxs[s])
        reads[c].wait()
        scats[c] = (pltpu.async_copy(bufs[s], xs_hbm.at[idxs[s][0]], sem_w),
                    pltpu.async_copy(bufs[s], xs_hbm.at[idxs[s][1]], sem_w))
    for c in (n_chunks - 2, n_chunks - 1):
        if c >= 0:
            scats[c][0].wait()
            scats[c][1].wait()


def _run_dispatch(x, dst0, dst1):
    mesh = plsc.VectorSubcoreMesh(core_axis_name="c", subcore_axis_name="s")
    f = pl.kernel(
        _dispatch_body,
        out_type=jax.ShapeDtypeStruct((PAD_TOTAL, D_MODEL), jnp.float32),
        mesh=mesh,
        scratch_types=[
            pltpu.VMEM((DISPATCH_CHUNK,), jnp.int32),
            pltpu.VMEM((DISPATCH_CHUNK,), jnp.int32),
            pltpu.VMEM((DISPATCH_CHUNK,), jnp.int32),
            pltpu.VMEM((DISPATCH_CHUNK,), jnp.int32),
            pltpu.VMEM((DISPATCH_CHUNK, D_MODEL), jnp.float32),
            pltpu.VMEM((DISPATCH_CHUNK, D_MODEL), jnp.float32),
            pltpu.SemaphoreType.DMA,
            pltpu.SemaphoreType.DMA,
            pltpu.SemaphoreType.DMA,
        ],
    )
    return f(x, dst0, dst1)


def _mm_body(te_ref, tv_ref, x_ref, wg_ref, wu_ref, wd_ref, y_ref):
    @pl.when(tv_ref[pl.program_id(0)] > 0)
    def _():
        x = x_ref[...]
        g = lax.dot_general(x, wg_ref[0], (((1,), (1,)), ((), ())),
                            preferred_element_type=jnp.float32)
        u = lax.dot_general(x, wu_ref[0], (((1,), (1,)), ((), ())),
                            preferred_element_type=jnp.float32)
        h = g * jax.nn.sigmoid(g) * u
        y_ref[...] = lax.dot_general(h, wd_ref[0], (((1,), (1,)), ((), ())),
                                     preferred_element_type=jnp.float32)


def _run_grouped_mm(x_sorted, w_gate, w_up, w_down, tile_expert, tile_valid):
    grid_spec = pltpu.PrefetchScalarGridSpec(
        num_scalar_prefetch=2,
        grid=(NT,),
        in_specs=[
            pl.BlockSpec((TM, D_MODEL), lambda i, te, tv: (i, 0)),
            pl.BlockSpec((1, D_FF, D_MODEL), lambda i, te, tv: (te[i], 0, 0)),
            pl.BlockSpec((1, D_FF, D_MODEL), lambda i, te, tv: (te[i], 0, 0)),
            pl.BlockSpec((1, D_MODEL, D_FF), lambda i, te, tv: (te[i], 0, 0)),
        ],
        out_specs=pl.BlockSpec((TM, D_MODEL), lambda i, te, tv: (i, 0)),
    )
    return pl.pallas_call(
        _mm_body,
        grid_spec=grid_spec,
        out_shape=jax.ShapeDtypeStruct((PAD_TOTAL, D_MODEL), jnp.float32),
        compiler_params=pltpu.CompilerParams(
            dimension_semantics=("arbitrary",),
            vmem_limit_bytes=100 * 1024 * 1024,
        ),
    )(tile_expert, tile_valid, x_sorted, w_gate, w_up, w_down)


def _combine_body(y_hbm, p0_hbm, p1_hbm, w0_hbm, w1_hbm, out_hbm,
                  i0_v, i1_v, w0_v, w1_v, y0_v, y1_v, ov_v, sem0, sem1):
    wid = lax.axis_index("s") * 2 + lax.axis_index("c")
    per_w = N_TOKENS // NW
    base = wid * per_w
    n_slice = D_MODEL // 16

    def chunk(ci, carry):
        b = base + ci * COMBINE_CHUNK
        pltpu.sync_copy(p0_hbm.at[pl.ds(b, COMBINE_CHUNK)], i0_v)
        pltpu.sync_copy(p1_hbm.at[pl.ds(b, COMBINE_CHUNK)], i1_v)
        pltpu.sync_copy(w0_hbm.at[pl.ds(b, COMBINE_CHUNK)], w0_v)
        pltpu.sync_copy(w1_hbm.at[pl.ds(b, COMBINE_CHUNK)], w1_v)
        cp0 = pltpu.async_copy(y_hbm.at[i0_v], y0_v, sem0)
        cp1 = pltpu.async_copy(y_hbm.at[i1_v], y1_v, sem1)
        cp0.wait()
        cp1.wait()

        def per_token(j, c2):
            wb0 = plsc.load_gather(w0_v, [jnp.full((16,), j, jnp.int32)])
            wb1 = plsc.load_gather(w1_v, [jnp.full((16,), j, jnp.int32)])

            def per_slice(s, c3):
                sl = pl.ds(s * 16, 16)
                ov_v[j, sl] = y0_v[j, sl] * wb0 + y1_v[j, sl] * wb1
                return c3

            lax.fori_loop(0, n_slice, per_slice, 0)
            return c2

        lax.fori_loop(0, COMBINE_CHUNK, per_token, 0)
        pltpu.sync_copy(ov_v, out_hbm.at[pl.ds(b, COMBINE_CHUNK)])
        return carry

    lax.fori_loop(0, per_w // COMBINE_CHUNK, chunk, 0)


def _run_combine(y_sorted, pos, topk_w):
    mesh = plsc.VectorSubcoreMesh(core_axis_name="c", subcore_axis_name="s")
    f = pl.kernel(
        _combine_body,
        out_type=jax.ShapeDtypeStruct((N_TOKENS, D_MODEL), jnp.float32),
        mesh=mesh,
        scratch_types=[
            pltpu.VMEM((COMBINE_CHUNK,), jnp.int32),
            pltpu.VMEM((COMBINE_CHUNK,), jnp.int32),
            pltpu.VMEM((COMBINE_CHUNK,), jnp.float32),
            pltpu.VMEM((COMBINE_CHUNK,), jnp.float32),
            pltpu.VMEM((COMBINE_CHUNK, D_MODEL), jnp.float32),
            pltpu.VMEM((COMBINE_CHUNK, D_MODEL), jnp.float32),
            pltpu.VMEM((COMBINE_CHUNK, D_MODEL), jnp.float32),
            pltpu.SemaphoreType.DMA,
            pltpu.SemaphoreType.DMA,
        ],
        compiler_params=pltpu.CompilerParams(needs_layout_passes=False),
    )
    p0 = pos[:, 0].astype(jnp.int32)
    p1 = pos[:, 1].astype(jnp.int32)
    w0 = topk_w[:, 0]
    w1 = topk_w[:, 1]
    return f(y_sorted, p0, p1, w0, w1)


def kernel(hidden_states, gate_w, w_gate, w_up, w_down):
    B, S, H = hidden_states.shape
    x = hidden_states.reshape(-1, H)
    topk_ids, topk_w = _run_router(x, gate_w)
    dst, tile_expert, tile_valid = _build_metadata(topk_ids)
    pos = dst.reshape(N_TOKENS, TOP_K)
    x_sorted = _run_dispatch(x, pos[:, 0], pos[:, 1])
    y_sorted = _run_grouped_mm(x_sorted, w_gate, w_up, w_down, tile_expert, tile_valid)
    out = _run_combine(y_sorted, pos, topk_w)
    return out.reshape(B, S, H)
